# baseline (speedup 1.0000x reference)
import functools

import jax
import jax.numpy as jnp
from jax import lax
from jax.experimental import pallas as pl
from jax.experimental.pallas import tpu as pltpu
from jax.experimental.pallas import tpu_sc as plsc

_INFO = plsc.get_sparse_core_info()
_NC = _INFO.num_cores
_NS = _INFO.num_subcores
_NW = _NC * _NS
_L = _INFO.num_lanes


@functools.partial(jax.jit, static_argnames=("batch", "seq", "d", "n"))
def _sc_gather(x2, index, *, batch, seq, d, n):
    total = batch * n
    rows_per_w = total // _NW

    mesh = plsc.VectorSubcoreMesh(core_axis_name="c", subcore_axis_name="s")

    @functools.partial(
        pl.kernel,
        mesh=mesh,
        out_type=jax.ShapeDtypeStruct((total, d), jnp.float32),
        scratch_types=[
            pltpu.VMEM((rows_per_w,), jnp.int32),
            pltpu.VMEM((rows_per_w,), jnp.int32),
            pltpu.VMEM((rows_per_w, d), jnp.float32),
            pltpu.SemaphoreType.DMA,
        ],
    )
    def k(x_hbm, idx_hbm, out_hbm, idx_v, rid_v, rows_v, sem):
        wid = lax.axis_index("s") * _NC + lax.axis_index("c")
        base = wid * rows_per_w
        b = base // n
        pos = base - b * n
        pltpu.sync_copy(idx_hbm.at[pl.ds(pos, rows_per_w)], idx_v)
        rid_v[...] = idx_v[...] + b * seq
        pltpu.async_copy(x_hbm.at[rid_v], rows_v, sem).wait()
        pltpu.sync_copy(rows_v, out_hbm.at[pl.ds(base, rows_per_w)])

    return k(x2, index)


def kernel(x, index):
    batch, seq, d = x.shape
    n = index.shape[0]
    x2 = x.reshape(batch * seq, d)
    out = _sc_gather(x2, index, batch=batch, seq=seq, d=d, n=n)
    return out.reshape(batch, n, d)
